# Initial kernel scaffold; baseline (speedup 1.0000x reference)
import jax
import jax.numpy as jnp
from jax.experimental import pallas as pl


def kernel(x, wordlist):
    raise NotImplementedError("write your pallas kernel here")


# speedup vs baseline: 2.2559x; 2.2559x over previous
import functools

import numpy as np
import jax
import jax.numpy as jnp
from jax import lax
from jax.experimental import pallas as pl
from jax.experimental.pallas import tpu as pltpu
from jax.experimental.pallas import tpu_sc as plsc

LANES = 16


@functools.lru_cache(maxsize=4)
def _pos_matrix_np(L: int, D: int) -> np.ndarray:
    jmax = (D - 1) // 2
    i = np.arange(L, dtype=np.float32)[:, None]
    j = np.arange(jmax, dtype=np.float32)[None, :]
    angle = (i / np.power(10000.0, 2.0 * j / np.float32(D))).astype(np.float32)
    pm = np.zeros((L, D), dtype=np.float32)
    pm[:, 0 : 2 * jmax : 2] = np.sin(angle)
    pm[:, 1 : 2 * jmax : 2] = np.cos(angle)
    return pm


def _sc_info():
    try:
        info = plsc.get_sparse_core_info()
        return info.num_cores, info.num_subcores
    except Exception:
        return 2, 16


@functools.lru_cache(maxsize=4)
def _build_kernel(L: int, V: int, D: int):
    NC, NS = _sc_info()
    NW = NC * NS
    b_per_w = L // NW
    C = 64
    n_chunks = b_per_w // C
    assert L % NW == 0 and b_per_w % C == 0 and D % LANES == 0

    mesh = plsc.VectorSubcoreMesh(core_axis_name="c", subcore_axis_name="s")

    @functools.partial(
        pl.kernel,
        mesh=mesh,
        out_type=jax.ShapeDtypeStruct((L, D), jnp.float32),
        scratch_types=[
            pltpu.VMEM((n_chunks, C), jnp.int32),
            pltpu.VMEM((C, D), jnp.float32),
            pltpu.VMEM((C, D), jnp.float32),
            pltpu.SemaphoreType.DMA,
        ],
    )
    def emb(x_hbm, table_hbm, pm_hbm, out_hbm, idx_v, rows_v, pm_v, sem):
        wid = lax.axis_index("s") * NC + lax.axis_index("c")
        base = wid * b_per_w
        pltpu.sync_copy(x_hbm.at[wid], idx_v)
        for ci in range(n_chunks):
            row0 = base + ci * C
            gather = pltpu.async_copy(table_hbm.at[idx_v.at[ci]], rows_v, sem)
            pltpu.sync_copy(pm_hbm.at[pl.ds(row0, C)], pm_v)
            gather.wait()

            def add_row(r, _):
                def add_vec(j, _):
                    s = pl.ds(pl.multiple_of(j * LANES, LANES), LANES)
                    rows_v[r, s] = rows_v[r, s] + pm_v[r, s]
                    return 0
                return lax.fori_loop(0, D // LANES, add_vec, 0, unroll=4)

            lax.fori_loop(0, C, add_row, 0)
            pltpu.sync_copy(rows_v, out_hbm.at[pl.ds(row0, C)])

    return emb


def kernel(x, wordlist):
    L = x.shape[0]
    V, D = wordlist.shape
    NC, NS = _sc_info()
    NW = NC * NS
    pm = jnp.asarray(_pos_matrix_np(L, D))
    emb = _build_kernel(L, V, D)
    x_grp = x.astype(jnp.int32).reshape(NW, L // NW // 64, 64)
    return emb(x_grp, wordlist, pm)


# speedup vs baseline: 2.3046x; 1.0216x over previous
import functools

import numpy as np
import jax
import jax.numpy as jnp
from jax import lax
from jax.experimental import pallas as pl
from jax.experimental.pallas import tpu as pltpu
from jax.experimental.pallas import tpu_sc as plsc

LANES = 16


@functools.lru_cache(maxsize=4)
def _pos_matrix_np(L: int, D: int) -> np.ndarray:
    jmax = (D - 1) // 2
    i = np.arange(L, dtype=np.float32)[:, None]
    j = np.arange(jmax, dtype=np.float32)[None, :]
    angle = (i / np.power(10000.0, 2.0 * j / np.float32(D))).astype(np.float32)
    pm = np.zeros((L, D), dtype=np.float32)
    pm[:, 0 : 2 * jmax : 2] = np.sin(angle)
    pm[:, 1 : 2 * jmax : 2] = np.cos(angle)
    return pm


def _sc_info():
    try:
        info = plsc.get_sparse_core_info()
        return info.num_cores, info.num_subcores
    except Exception:
        return 2, 16


_CHUNK = 32


@functools.lru_cache(maxsize=4)
def _build_kernel(L: int, V: int, D: int):
    NC, NS = _sc_info()
    NW = NC * NS
    b_per_w = L // NW
    C = _CHUNK
    n_chunks = b_per_w // C
    assert L % NW == 0 and b_per_w % C == 0 and D % LANES == 0

    mesh = plsc.VectorSubcoreMesh(core_axis_name="c", subcore_axis_name="s")

    @functools.partial(
        pl.kernel,
        mesh=mesh,
        out_type=jax.ShapeDtypeStruct((L, D), jnp.float32),
        scratch_types=[
            pltpu.VMEM((n_chunks, C), jnp.int32),
            pltpu.VMEM((2, C, D), jnp.float32),
            pltpu.VMEM((2, C, D), jnp.float32),
            pltpu.SemaphoreType.DMA((2,)),
            pltpu.SemaphoreType.DMA((2,)),
            pltpu.SemaphoreType.DMA((2,)),
        ],
    )
    def emb(x_hbm, table_hbm, pm_hbm, out_hbm, idx_v, rows_v, pm_v,
            gsem, psem, osem):
        wid = lax.axis_index("s") * NC + lax.axis_index("c")
        base = wid * b_per_w
        pltpu.sync_copy(x_hbm.at[wid], idx_v)

        def fire(ci):
            slot = ci % 2
            row0 = base + ci * C
            g = pltpu.async_copy(table_hbm.at[idx_v.at[ci]],
                                 rows_v.at[slot], gsem.at[slot])
            p = pltpu.async_copy(pm_hbm.at[pl.ds(row0, C)],
                                 pm_v.at[slot], psem.at[slot])
            return g, p

        inflight = {0: fire(0)}
        out_cp = {}
        for ci in range(n_chunks):
            slot = ci % 2
            if ci + 1 < n_chunks:
                if ci >= 1:
                    out_cp[ci - 1].wait()
                inflight[ci + 1] = fire(ci + 1)
            g, p = inflight.pop(ci)
            g.wait()
            p.wait()

            def add_row(r, _):
                def add_vec(j, _):
                    s = pl.ds(pl.multiple_of(j * LANES, LANES), LANES)
                    plsc.addupdate(rows_v.at[slot, r, s], pm_v[slot, r, s])
                    return 0
                return lax.fori_loop(0, D // LANES, add_vec, 0, unroll=8)

            lax.fori_loop(0, C, add_row, 0)
            row0 = base + ci * C
            out_cp[ci] = pltpu.async_copy(rows_v.at[slot],
                                          out_hbm.at[pl.ds(row0, C)],
                                          osem.at[slot])
        out_cp[n_chunks - 2].wait()
        out_cp[n_chunks - 1].wait()

    return emb


def kernel(x, wordlist):
    L = x.shape[0]
    V, D = wordlist.shape
    NC, NS = _sc_info()
    NW = NC * NS
    pm = jnp.asarray(_pos_matrix_np(L, D))
    emb = _build_kernel(L, V, D)
    x_grp = x.astype(jnp.int32).reshape(NW, L // NW // _CHUNK, _CHUNK)
    return emb(x_grp, wordlist, pm)
